# Initial kernel scaffold; baseline (speedup 1.0000x reference)
import jax
import jax.numpy as jnp
from jax.experimental import pallas as pl


def kernel(x, edge_attr, edge_length, edge_src, edge_dst, fc_w1, fc_w2, sc_ws, sc_wv):
    raise NotImplementedError("write your pallas kernel here")


# speedup vs baseline: 1.4315x; 1.4315x over previous
import functools
import math

import jax
import jax.numpy as jnp
import numpy as np
from jax import lax
from jax.experimental import pallas as pl
from jax.experimental.pallas import tpu as pltpu

N = 10000
E = 160000
NS = 16
NV = 8
NR = 8
WNE = 896
ALPHA = 1.0 / math.sqrt(24.0)

E_PAD = 163840
N_PAD = 10240
N_ACC = N + 16

EB = 1024
NB = 1024


def _build_rs():
    r = np.zeros((64, WNE), dtype=np.float32)
    s = np.zeros((WNE, 64), dtype=np.float32)
    for i in range(NS):
        for j in range(NS):
            c = i * 16 + j
            r[i, c] = 1.0
            s[c, j] = ALPHA
    for a in range(NV):
        for j in range(NS):
            c = 256 + a * 16 + j
            r[16 + a, c] = 1.0
            s[c, j] = ALPHA
    for i in range(NS):
        for a in range(NV):
            c = 384 + i * 8 + a
            r[i, c] = 1.0
            s[c, 16 + a] = ALPHA
    for b in range(NV):
        for a in range(NV):
            c = 512 + b * 8 + a
            r[16 + b, c] = 1.0
            s[c, 16 + a] = ALPHA
    for i in range(NS):
        for a in range(NV):
            c = 576 + i * 8 + a
            r[24 + i, c] = 1.0
            s[c, 48 + a] = ALPHA
    for k in range(3):
        for b in range(NV):
            for a in range(NV):
                c = 704 + k * 64 + b * 8 + a
                r[40 + 8 * k + b, c] = 1.0
                s[c, 24 + 8 * k + a] = ALPHA
    return r, s


_R_MAT, _S_MAT = _build_rs()
_INV_SQRT3 = 1.0 / math.sqrt(3.0)


def _edge_kernel(eb_ref, xj_ref, w1_ref, w2e_ref, r_ref, s_ref, out_ref):
    eb = eb_ref[...]
    xj = xj_ref[...]
    el = eb[:, 4:5]
    centers = lax.broadcasted_iota(
        jnp.int32, (1, NR), 1).astype(jnp.float32) * (5.0 / 7.0)
    radial = jnp.exp(-0.5 * (el - centers) ** 2)
    h = jnp.dot(radial, w1_ref[...], preferred_element_type=jnp.float32)
    h = h * jax.nn.sigmoid(h)
    w = jnp.dot(h, w2e_ref[...], preferred_element_type=jnp.float32)

    y0 = eb[:, 0:1]
    sy = xj[:, 0:16] * y0
    vx = xj[:, 16:24]
    vy = xj[:, 24:32]
    vz = xj[:, 32:40]
    d = (vx * eb[:, 1:2] + vy * eb[:, 2:3] + vz * eb[:, 3:4]) * _INV_SQRT3
    u = jnp.concatenate(
        [sy, d, xj[:, 0:16], vx * y0, vy * y0, vz * y0], axis=1)
    f = jnp.dot(u, r_ref[...], preferred_element_type=jnp.float32)
    mp = jnp.dot(w * f, s_ref[...], preferred_element_type=jnp.float32)

    t5 = mp[:, 48:56]
    vecx = mp[:, 24:32] + t5 * eb[:, 1:2]
    vecy = mp[:, 32:40] + t5 * eb[:, 2:3]
    vecz = mp[:, 40:48] + t5 * eb[:, 3:4]
    cnt = jnp.where(
        lax.broadcasted_iota(jnp.int32, (eb.shape[0], 16), 1) == 0, 1.0, 0.0)
    out_ref[...] = jnp.concatenate(
        [mp[:, 0:24], vecx, vecy, vecz, cnt], axis=1)


def _edge_stage(eb, xj, w1s, w2e, r_mat, s_mat):
    grid = E_PAD // EB
    return pl.pallas_call(
        _edge_kernel,
        grid=(grid,),
        in_specs=[
            pl.BlockSpec((EB, 8), lambda i: (i, 0)),
            pl.BlockSpec((EB, 64), lambda i: (i, 0)),
            pl.BlockSpec((NR, 64), lambda i: (0, 0)),
            pl.BlockSpec((64, WNE), lambda i: (0, 0)),
            pl.BlockSpec((64, WNE), lambda i: (0, 0)),
            pl.BlockSpec((WNE, 64), lambda i: (0, 0)),
        ],
        out_specs=pl.BlockSpec((EB, 64), lambda i: (i, 0)),
        out_shape=jax.ShapeDtypeStruct((E_PAD, 64), jnp.float32),
    )(eb, xj, w1s, w2e, r_mat, s_mat)


def _node_kernel(ma_ref, mb_ref, x_ref, ws_ref, wvt_ref, out_ref):
    m = ma_ref[...] + mb_ref[...]
    x = x_ref[...]
    inv = 1.0 / jnp.maximum(m[:, 48:49], 1.0)
    gs = m[:, 0:16] * inv
    gs = gs * jax.nn.sigmoid(gs)
    gg = jax.nn.sigmoid(m[:, 16:24] * inv)
    sc_s = jnp.dot(x[:, 0:16], ws_ref[...],
                   preferred_element_type=jnp.float32) * 0.25
    wvt = wvt_ref[...]
    isq8 = 1.0 / math.sqrt(8.0)
    outs = gs + sc_s
    ox = m[:, 24:32] * inv * gg + jnp.dot(
        x[:, 16:24], wvt, preferred_element_type=jnp.float32) * isq8
    oy = m[:, 32:40] * inv * gg + jnp.dot(
        x[:, 24:32], wvt, preferred_element_type=jnp.float32) * isq8
    oz = m[:, 40:48] * inv * gg + jnp.dot(
        x[:, 32:40], wvt, preferred_element_type=jnp.float32) * isq8
    nv = jnp.sqrt(ox * ox + oy * oy + oz * oz)
    out_ref[...] = jnp.concatenate([jnp.abs(outs), nv], axis=1)


def _node_stage(ma, mb, x_pad, sc_ws, sc_wv_t):
    grid = N_PAD // NB
    return pl.pallas_call(
        _node_kernel,
        grid=(grid,),
        in_specs=[
            pl.BlockSpec((NB, 64), lambda i: (i, 0)),
            pl.BlockSpec((NB, 64), lambda i: (i, 0)),
            pl.BlockSpec((NB, 64), lambda i: (i, 0)),
            pl.BlockSpec((NS, NS), lambda i: (0, 0)),
            pl.BlockSpec((NV, NV), lambda i: (0, 0)),
        ],
        out_specs=pl.BlockSpec((NB, 24), lambda i: (i, 0)),
        out_shape=jax.ShapeDtypeStruct((N_PAD, 24), jnp.float32),
    )(ma, mb, x_pad, sc_ws, sc_wv_t)


def kernel(x, edge_attr, edge_length, edge_src, edge_dst,
           fc_w1, fc_w2, sc_ws, sc_wv):
    s = x[:, :NS]
    v = x[:, NS:].reshape(N, NV, 3)
    x_pad = jnp.concatenate(
        [s, v[:, :, 0], v[:, :, 1], v[:, :, 2],
         jnp.zeros((N, 24), jnp.float32)], axis=1)

    eb = jnp.concatenate(
        [edge_attr, edge_length[:, None], jnp.zeros((E, 3), jnp.float32)],
        axis=1)
    eb = jnp.pad(eb, ((0, E_PAD - E), (0, 0)))
    src = jnp.pad(edge_src, (0, E_PAD - E))
    dst = jnp.pad(edge_dst, (0, E_PAD - E), constant_values=N)

    w1s = fc_w1 / math.sqrt(float(NR))
    w2e = jnp.concatenate(
        [fc_w2, fc_w2[:, 704:768], fc_w2[:, 704:768]], axis=1) / 8.0
    r_mat = jnp.asarray(_R_MAT)
    s_mat = jnp.asarray(_S_MAT)

    xj = jnp.take(x_pad, src, axis=0)

    m = _edge_stage(eb, xj, w1s, w2e, r_mat, s_mat)

    msum = jax.ops.segment_sum(m, dst, num_segments=N_ACC)

    ma = jnp.pad(msum[:N], ((0, N_PAD - N), (0, 0)))
    mb = jnp.zeros_like(ma)
    x_pad_n = jnp.pad(x_pad, ((0, N_PAD - N), (0, 0)))
    out = _node_stage(ma, mb, x_pad_n, sc_ws, sc_wv.T)
    return out[:N]
